# baseline (speedup 1.0000x reference)
import functools

import jax
import jax.numpy as jnp
from jax.experimental import pallas as pl
from jax.experimental.pallas import tpu as pltpu

_BLK = 512


def _mm1_kernel(adj_ref, e_ref, o_ref):
    y = jnp.dot(adj_ref[...], e_ref[...], preferred_element_type=jnp.float32)
    o_ref[...] = 4.0 * jnp.maximum(y, 0.0)


def _mm2_kernel(inter_ref, adj_ref, e_ref, t1_ref, o_ref, *, half):
    i = pl.program_id(0)
    y = jnp.dot(adj_ref[...], e_ref[...], preferred_element_type=jnp.float32)
    t2 = 4.0 * jnp.maximum(y, 0.0)

    @pl.when(i < half)
    def _():
        o_ref[...] = t2

    @pl.when(i >= half)
    def _():
        w = inter_ref[0]
        o_ref[...] = w * t1_ref[...] + (1.0 - w) * t2


def kernel(adj1, adj2, dEmbed, mEmbed, pEmbed, inter):
    e1 = jnp.concatenate([dEmbed, mEmbed], axis=0)
    e2 = jnp.concatenate([pEmbed, mEmbed], axis=0)
    n = adj1.shape[0]
    d = dEmbed.shape[0]
    p = pEmbed.shape[0]
    f = dEmbed.shape[1]
    blk = _BLK
    grid = n // blk
    half = d // blk

    o1 = pl.pallas_call(
        _mm1_kernel,
        grid=(grid,),
        in_specs=[
            pl.BlockSpec((blk, n), lambda i: (i, 0)),
            pl.BlockSpec((n, f), lambda i: (0, 0)),
        ],
        out_specs=pl.BlockSpec((blk, f), lambda i: (i, 0)),
        out_shape=jax.ShapeDtypeStruct((n, f), jnp.float32),
    )(adj1, e1)

    o2 = pl.pallas_call(
        functools.partial(_mm2_kernel, half=half),
        grid=(grid,),
        in_specs=[
            pl.BlockSpec(memory_space=pltpu.SMEM),
            pl.BlockSpec((blk, n), lambda i: (i, 0)),
            pl.BlockSpec((n, f), lambda i: (0, 0)),
            pl.BlockSpec((blk, f), lambda i: (i, 0)),
        ],
        out_specs=pl.BlockSpec((blk, f), lambda i: (i, 0)),
        out_shape=jax.ShapeDtypeStruct((n, f), jnp.float32),
    )(inter, adj2, e2, o1)
    return (o2[p:], o1[:d], o2[:p])
